# speedup vs baseline: 7.3039x; 7.3039x over previous
import math

import jax
import jax.numpy as jnp
from jax.experimental import pallas as pl

VOC = 32000
N_ROWS = 2048
PAD = 0
SMOOTH = 0.1
CONF = 1.0 - SMOOTH
VALUE = SMOOTH / (VOC - 2)
C1 = (VOC - 2) * VALUE * math.log(VALUE) + CONF * math.log(CONF)

ROW_BLK = 512
COL_BLK = 6400
GR = N_ROWS // ROW_BLK
GC = VOC // COL_BLK


def _body(x_ref, t_ref, o_ref):
    r = pl.program_id(0)
    v = pl.program_id(1)

    @pl.when(jnp.logical_and(r == 0, v == 0))
    def _init():
        o_ref[...] = jnp.zeros((1, 1), jnp.float32)

    xb = x_ref[...]
    tb = t_ref[...]
    nonpad = tb != PAD
    cols = jax.lax.broadcasted_iota(jnp.int32, (ROW_BLK, COL_BLK), 1) + v * COL_BLK
    w = jnp.where(cols == tb, -CONF, jnp.where(cols == 0, 0.0, -VALUE))
    w = jnp.where(nonpad, w, 0.0)
    acc = jnp.sum(w * xb)

    @pl.when(v == 0)
    def _const():
        cnt = jnp.sum(jnp.where(nonpad, 1.0, 0.0))
        o_ref[...] = o_ref[...] + jnp.reshape(cnt * C1, (1, 1))

    o_ref[...] = o_ref[...] + jnp.reshape(acc, (1, 1))


def kernel(x, target):
    t2 = target.reshape(N_ROWS, 1)
    out = pl.pallas_call(
        _body,
        grid=(GR, GC),
        in_specs=[
            pl.BlockSpec((ROW_BLK, COL_BLK), lambda r, v: (r, v)),
            pl.BlockSpec((ROW_BLK, 1), lambda r, v: (r, 0)),
        ],
        out_specs=pl.BlockSpec((1, 1), lambda r, v: (0, 0)),
        out_shape=jax.ShapeDtypeStruct((1, 1), jnp.float32),
    )(x, t2)
    return out[0, 0]
